# baseline (speedup 1.0000x reference)
import functools

import jax
import jax.numpy as jnp
from jax import lax
from jax.experimental import pallas as pl
from jax.experimental.pallas import tpu as pltpu
from jax.experimental.pallas import tpu_sc as plsc

B = 16384
D = 128
NC = 2
NS = 16
NW = NC * NS
BPW = B // NW
CHUNK = 128
NCHUNK = BPW // CHUNK
NGRP = CHUNK // 16


def _mf_body(users_hbm, items_hbm, ut_hbm, it_hbm, wb_hbm, out_hbm,
             uidx, iidx, u0, u1, i0, i1, wb_v, out_v, usem, isem):
    wid = lax.axis_index("s") * NC + lax.axis_index("c")
    base = wid * BPW

    pltpu.sync_copy(users_hbm.at[pl.ds(base, BPW)], uidx)
    pltpu.sync_copy(items_hbm.at[pl.ds(base, BPW)], iidx)
    pltpu.sync_copy(wb_hbm, wb_v)

    ubufs = (u0, u1)
    ibufs = (i0, i1)

    def start(c):
        ub = ubufs[c & 1]
        ib = ibufs[c & 1]
        hu = pltpu.async_copy(ut_hbm.at[uidx.at[pl.ds(c * CHUNK, CHUNK)]], ub, usem)
        hi = pltpu.async_copy(it_hbm.at[iidx.at[pl.ds(c * CHUNK, CHUNK)]], ib, isem)
        return hu, hi

    pending = start(0)

    riota = lax.iota(jnp.int32, 16)
    ridx = [riota + (g * 16) for g in range(NGRP)]
    bvec = wb_v[D]

    for c in range(NCHUNK):
        hu, hi = pending
        hu.wait()
        hi.wait()
        if c + 1 < NCHUNK:
            pending = start(c + 1)

        ub = ubufs[c & 1]
        ib = ibufs[c & 1]

        def dbody(d, accs, ub=ub, ib=ib):
            w_d = wb_v[d]
            col = (riota + d) & (D - 1)
            out = []
            for g in range(NGRP):
                ug = plsc.load_gather(ub, [ridx[g], col])
                ig = plsc.load_gather(ib, [ridx[g], col])
                pb = plsc.bitcast(ug * ig, jnp.int32)
                lsb = lax.shift_right_logical(pb, 16) & 1
                pb = (pb + 0x7FFF + lsb) & jnp.int32(-65536)
                p = plsc.bitcast(pb, jnp.float32)
                out.append(accs[g] + p * w_d)
            return tuple(out)

        accs = lax.fori_loop(
            0, D, dbody,
            tuple(jnp.zeros((16,), jnp.float32) for _ in range(NGRP)))

        for g in range(NGRP):
            out_v[pl.ds(c * CHUNK + g * 16, 16)] = accs[g] + bvec

    pltpu.sync_copy(out_v, out_hbm.at[pl.ds(base, BPW)])


@jax.jit
def kernel(users, items, user_table, item_table, W, b):
    users_r = users.astype(jnp.int32)
    items_r = items.astype(jnp.int32)
    wi = lax.bitcast_convert_type(W.astype(jnp.float32), jnp.int32)
    wi = (wi + 0x7FFF + (lax.shift_right_logical(wi, 16) & 1)) & jnp.int32(-65536)
    w_rounded = lax.bitcast_convert_type(wi, jnp.float32)[:, 0]
    skew = (jnp.arange(D)[:, None] + jnp.arange(16)[None, :]) % D
    wsk = w_rounded[skew]
    wb = jnp.concatenate([wsk, jnp.broadcast_to(b, (1, 16))]).astype(jnp.float32)

    mesh = plsc.VectorSubcoreMesh(core_axis_name="c", subcore_axis_name="s")
    out = pl.kernel(
        _mf_body,
        mesh=mesh,
        compiler_params=pltpu.CompilerParams(needs_layout_passes=False),
        out_type=jax.ShapeDtypeStruct((B,), jnp.float32),
        scratch_types=[
            pltpu.VMEM((BPW,), jnp.int32),
            pltpu.VMEM((BPW,), jnp.int32),
            pltpu.VMEM((CHUNK, D), jnp.float32),
            pltpu.VMEM((CHUNK, D), jnp.float32),
            pltpu.VMEM((CHUNK, D), jnp.float32),
            pltpu.VMEM((CHUNK, D), jnp.float32),
            pltpu.VMEM((D + 1, 16), jnp.float32),
            pltpu.VMEM((BPW,), jnp.float32),
            pltpu.SemaphoreType.DMA,
            pltpu.SemaphoreType.DMA,
        ],
    )(users_r, items_r, user_table, item_table, wb)
    return out.reshape(B, 1)


# speedup vs baseline: 1.0063x; 1.0063x over previous
import functools

import jax
import jax.numpy as jnp
from jax import lax
from jax.experimental import pallas as pl
from jax.experimental.pallas import tpu as pltpu
from jax.experimental.pallas import tpu_sc as plsc

B = 16384
D = 128
NC = 2
NS = 16
NW = NC * NS
BPW = B // NW
CHUNK = 128
NCHUNK = BPW // CHUNK
NGRP = CHUNK // 16


def _mf_body(users_hbm, items_hbm, ut_hbm, it_hbm, wb_hbm, out_hbm,
             uidx, iidx, u0, u1, u2, i0, i1, i2, wb_v, out_v, usem, isem):
    wid = lax.axis_index("s") * NC + lax.axis_index("c")
    base = wid * BPW

    pltpu.sync_copy(users_hbm.at[pl.ds(base, BPW)], uidx)
    pltpu.sync_copy(items_hbm.at[pl.ds(base, BPW)], iidx)
    pltpu.sync_copy(wb_hbm, wb_v)

    ubufs = (u0, u1, u2)
    ibufs = (i0, i1, i2)

    def start(c):
        ub = ubufs[c % 3]
        ib = ibufs[c % 3]
        hu = pltpu.async_copy(ut_hbm.at[uidx.at[pl.ds(c * CHUNK, CHUNK)]], ub, usem)
        hi = pltpu.async_copy(it_hbm.at[iidx.at[pl.ds(c * CHUNK, CHUNK)]], ib, isem)
        return hu, hi

    pending = [start(0), start(1), start(2)]

    riota = lax.iota(jnp.int32, 16)
    ridx = [riota + (g * 16) for g in range(NGRP)]
    bvec = wb_v[D]

    for c in range(NCHUNK):
        hu, hi = pending[c]
        hu.wait()
        hi.wait()
        if c + 3 < NCHUNK:
            pending.append(start(c + 3))

        ub = ubufs[c % 3]
        ib = ibufs[c % 3]

        def dbody(d, accs, ub=ub, ib=ib):
            w_d = wb_v[d]
            col = (riota + d) & (D - 1)
            out = []
            for g in range(NGRP):
                ug = plsc.load_gather(ub, [ridx[g], col])
                ig = plsc.load_gather(ib, [ridx[g], col])
                pb = plsc.bitcast(ug * ig, jnp.int32)
                pb = (pb + 0x8000) & jnp.int32(-65536)
                p = plsc.bitcast(pb, jnp.float32)
                out.append(accs[g] + p * w_d)
            return tuple(out)

        accs = lax.fori_loop(
            0, D, dbody,
            tuple(jnp.zeros((16,), jnp.float32) for _ in range(NGRP)))

        for g in range(NGRP):
            out_v[pl.ds(c * CHUNK + g * 16, 16)] = accs[g] + bvec

    pltpu.sync_copy(out_v, out_hbm.at[pl.ds(base, BPW)])


@jax.jit
def kernel(users, items, user_table, item_table, W, b):
    users_r = users.astype(jnp.int32)
    items_r = items.astype(jnp.int32)
    wi = lax.bitcast_convert_type(W.astype(jnp.float32), jnp.int32)
    wi = (wi + 0x7FFF + (lax.shift_right_logical(wi, 16) & 1)) & jnp.int32(-65536)
    w_rounded = lax.bitcast_convert_type(wi, jnp.float32)[:, 0]
    skew = (jnp.arange(D)[:, None] + jnp.arange(16)[None, :]) % D
    wsk = w_rounded[skew]
    wb = jnp.concatenate([wsk, jnp.broadcast_to(b, (1, 16))]).astype(jnp.float32)

    mesh = plsc.VectorSubcoreMesh(core_axis_name="c", subcore_axis_name="s")
    out = pl.kernel(
        _mf_body,
        mesh=mesh,
        compiler_params=pltpu.CompilerParams(needs_layout_passes=False),
        out_type=jax.ShapeDtypeStruct((B,), jnp.float32),
        scratch_types=[
            pltpu.VMEM((BPW,), jnp.int32),
            pltpu.VMEM((BPW,), jnp.int32),
            pltpu.VMEM((CHUNK, D), jnp.float32),
            pltpu.VMEM((CHUNK, D), jnp.float32),
            pltpu.VMEM((CHUNK, D), jnp.float32),
            pltpu.VMEM((CHUNK, D), jnp.float32),
            pltpu.VMEM((CHUNK, D), jnp.float32),
            pltpu.VMEM((CHUNK, D), jnp.float32),
            pltpu.VMEM((D + 1, 16), jnp.float32),
            pltpu.VMEM((BPW,), jnp.float32),
            pltpu.SemaphoreType.DMA,
            pltpu.SemaphoreType.DMA,
        ],
    )(users_r, items_r, user_table, item_table, wb)
    return out.reshape(B, 1)


# speedup vs baseline: 1.0131x; 1.0067x over previous
import functools

import jax
import jax.numpy as jnp
from jax import lax
from jax.experimental import pallas as pl
from jax.experimental.pallas import tpu as pltpu
from jax.experimental.pallas import tpu_sc as plsc

B = 16384
D = 128
NC = 2
NS = 16
NW = NC * NS
BPW = B // NW
CHUNK = 128
NCHUNK = BPW // CHUNK
NGRP = CHUNK // 16


def _mf_body(users_hbm, items_hbm, ut_hbm, it_hbm, wb_hbm, out_hbm,
             uidx, iidx, u0, u1, u2, i0, i1, i2, wb_v, out_v, usem, isem):
    wid = lax.axis_index("s") * NC + lax.axis_index("c")
    base = wid * BPW

    pltpu.sync_copy(users_hbm.at[pl.ds(base, BPW)], uidx)
    pltpu.sync_copy(items_hbm.at[pl.ds(base, BPW)], iidx)
    pltpu.sync_copy(wb_hbm, wb_v)

    ubufs = (u0, u1, u2)
    ibufs = (i0, i1, i2)

    def start(c):
        ub = ubufs[c % 3]
        ib = ibufs[c % 3]
        hu = pltpu.async_copy(ut_hbm.at[uidx.at[pl.ds(c * CHUNK, CHUNK)]], ub, usem)
        hi = pltpu.async_copy(it_hbm.at[iidx.at[pl.ds(c * CHUNK, CHUNK)]], ib, isem)
        return hu, hi

    pending = [start(0), start(1), start(2)]

    riota = lax.iota(jnp.int32, 16)
    ridx = [riota + (g * 16) for g in range(NGRP)]
    bvec = wb_v[D]

    for c in range(NCHUNK):
        hu, hi = pending[c]
        hu.wait()
        hi.wait()

        ub = ubufs[c % 3]
        ib = ibufs[c % 3]

        def dbody(d, accs, ub=ub, ib=ib):
            w_d = wb_v[d]
            col = (riota + d) & (D - 1)
            out = []
            for g in range(NGRP):
                ug = plsc.load_gather(ub, [ridx[g], col])
                ig = plsc.load_gather(ib, [ridx[g], col])
                pb = plsc.bitcast(ug * ig, jnp.int32)
                pb = (pb + 0x8000) & jnp.int32(-65536)
                p = plsc.bitcast(pb, jnp.float32)
                out.append(accs[g] + p * w_d)
            return tuple(out)

        accs = lax.fori_loop(
            0, D, dbody,
            tuple(jnp.zeros((16,), jnp.float32) for _ in range(NGRP)))

        for g in range(NGRP):
            out_v[pl.ds(c * CHUNK + g * 16, 16)] = accs[g] + bvec

        if c + 3 < NCHUNK:
            pending.append(start(c + 3))

    pltpu.sync_copy(out_v, out_hbm.at[pl.ds(base, BPW)])


@jax.jit
def kernel(users, items, user_table, item_table, W, b):
    users_r = users.astype(jnp.int32)
    items_r = items.astype(jnp.int32)
    wi = lax.bitcast_convert_type(W.astype(jnp.float32), jnp.int32)
    wi = (wi + 0x7FFF + (lax.shift_right_logical(wi, 16) & 1)) & jnp.int32(-65536)
    w_rounded = lax.bitcast_convert_type(wi, jnp.float32)[:, 0]
    skew = (jnp.arange(D)[:, None] + jnp.arange(16)[None, :]) % D
    wsk = w_rounded[skew]
    wb = jnp.concatenate([wsk, jnp.broadcast_to(b, (1, 16))]).astype(jnp.float32)

    mesh = plsc.VectorSubcoreMesh(core_axis_name="c", subcore_axis_name="s")
    out = pl.kernel(
        _mf_body,
        mesh=mesh,
        compiler_params=pltpu.CompilerParams(needs_layout_passes=False),
        out_type=jax.ShapeDtypeStruct((B,), jnp.float32),
        scratch_types=[
            pltpu.VMEM((BPW,), jnp.int32),
            pltpu.VMEM((BPW,), jnp.int32),
            pltpu.VMEM((CHUNK, D), jnp.float32),
            pltpu.VMEM((CHUNK, D), jnp.float32),
            pltpu.VMEM((CHUNK, D), jnp.float32),
            pltpu.VMEM((CHUNK, D), jnp.float32),
            pltpu.VMEM((CHUNK, D), jnp.float32),
            pltpu.VMEM((CHUNK, D), jnp.float32),
            pltpu.VMEM((D + 1, 16), jnp.float32),
            pltpu.VMEM((BPW,), jnp.float32),
            pltpu.SemaphoreType.DMA,
            pltpu.SemaphoreType.DMA,
        ],
    )(users_r, items_r, user_table, item_table, wb)
    return out.reshape(B, 1)


# speedup vs baseline: 1.4829x; 1.4637x over previous
import functools

import jax
import jax.numpy as jnp
from jax import lax
from jax.experimental import pallas as pl
from jax.experimental.pallas import tpu as pltpu
from jax.experimental.pallas import tpu_sc as plsc

B = 16384
D = 128
NC = 2
NS = 16
NW = NC * NS
BPW = B // NW
CHUNK = 128
NCHUNK = BPW // CHUNK
NGRP = CHUNK // 16


def _mf_body(users_hbm, items_hbm, ut_hbm, it_hbm, wb_hbm, out_hbm,
             uidx, iidx, u0, u1, u2, i0, i1, i2, wb_v, out_v, usem, isem):
    wid = lax.axis_index("s") * NC + lax.axis_index("c")
    base = wid * BPW

    pltpu.sync_copy(users_hbm.at[pl.ds(base, BPW)], uidx)
    pltpu.sync_copy(items_hbm.at[pl.ds(base, BPW)], iidx)
    pltpu.sync_copy(wb_hbm, wb_v)

    ubufs = (u0, u1, u2)
    ibufs = (i0, i1, i2)

    def start(c):
        ub = ubufs[c % 3]
        ib = ibufs[c % 3]
        hu = pltpu.async_copy(ut_hbm.at[uidx.at[pl.ds(c * CHUNK, CHUNK)]], ub, usem)
        hi = pltpu.async_copy(it_hbm.at[iidx.at[pl.ds(c * CHUNK, CHUNK)]], ib, isem)
        return hu, hi

    pending = [start(0), start(1), start(2)]

    riota = lax.iota(jnp.int32, 16)
    ridx = [riota + (g * 16) for g in range(NGRP)]
    bvec = wb_v[D]

    for c in range(NCHUNK):
        hu, hi = pending[c]
        hu.wait()
        hi.wait()

        ub = ubufs[c % 3]
        ib = ibufs[c % 3]

        def dbody(d, accs, ub=ub, ib=ib):
            w_d = wb_v[d]
            col = (riota + d) & (D - 1)
            out = []
            for g in range(NGRP):
                ug = plsc.load_gather(ub, [ridx[g], col])
                ig = plsc.load_gather(ib, [ridx[g], col])
                pb = plsc.bitcast(ug * ig, jnp.int32)
                pb = (pb + 0x8000) & jnp.int32(-65536)
                p = plsc.bitcast(pb, jnp.float32)
                out.append(accs[g] + p * w_d)
            return tuple(out)

        accs = lax.fori_loop(
            0, D, dbody,
            tuple(jnp.zeros((16,), jnp.float32) for _ in range(NGRP)))

        for g in range(NGRP):
            out_v[pl.ds(c * CHUNK + g * 16, 16)] = accs[g] + bvec

        if c + 3 < NCHUNK:
            pending.append(start(c + 3))

    pltpu.sync_copy(out_v, out_hbm.at[pl.ds(base, BPW)])


@jax.jit
def kernel(users, items, user_table, item_table, W, b):
    users_r = users.astype(jnp.int32)
    items_r = items.astype(jnp.int32)
    wi = lax.bitcast_convert_type(W.astype(jnp.float32), jnp.int32)
    wi = (wi + 0x7FFF + (lax.shift_right_logical(wi, 16) & 1)) & jnp.int32(-65536)
    w_rounded = lax.bitcast_convert_type(wi, jnp.float32)[:, 0]
    w2 = jnp.concatenate([w_rounded, w_rounded])
    wsk = jnp.stack([w2[l:l + D] for l in range(16)], axis=1)
    wb = jnp.concatenate([wsk, jnp.broadcast_to(b, (1, 16))]).astype(jnp.float32)

    mesh = plsc.VectorSubcoreMesh(core_axis_name="c", subcore_axis_name="s")
    out = pl.kernel(
        _mf_body,
        mesh=mesh,
        compiler_params=pltpu.CompilerParams(needs_layout_passes=False),
        out_type=jax.ShapeDtypeStruct((B,), jnp.float32),
        scratch_types=[
            pltpu.VMEM((BPW,), jnp.int32),
            pltpu.VMEM((BPW,), jnp.int32),
            pltpu.VMEM((CHUNK, D), jnp.float32),
            pltpu.VMEM((CHUNK, D), jnp.float32),
            pltpu.VMEM((CHUNK, D), jnp.float32),
            pltpu.VMEM((CHUNK, D), jnp.float32),
            pltpu.VMEM((CHUNK, D), jnp.float32),
            pltpu.VMEM((CHUNK, D), jnp.float32),
            pltpu.VMEM((D + 1, 16), jnp.float32),
            pltpu.VMEM((BPW,), jnp.float32),
            pltpu.SemaphoreType.DMA,
            pltpu.SemaphoreType.DMA,
        ],
    )(users_r, items_r, user_table, item_table, wb)
    return out.reshape(B, 1)


# speedup vs baseline: 1.5478x; 1.0438x over previous
import functools

import jax
import jax.numpy as jnp
from jax import lax
from jax.experimental import pallas as pl
from jax.experimental.pallas import tpu as pltpu
from jax.experimental.pallas import tpu_sc as plsc

B = 16384
D = 128
NC = 2
NS = 16
NW = NC * NS
BPW = B // NW
CHUNK = 128
NCHUNK = BPW // CHUNK
NGRP = CHUNK // 16


def _mf_body(users_hbm, items_hbm, ut_hbm, it_hbm, wb_hbm, out_hbm,
             uidx, iidx, u0, u1, u2, i0, i1, i2, wb_v, out_v, usem, isem):
    wid = lax.axis_index("s") * NC + lax.axis_index("c")
    base = wid * BPW

    h_u = pltpu.async_copy(users_hbm.at[pl.ds(base, BPW)], uidx, usem)
    h_i = pltpu.async_copy(items_hbm.at[pl.ds(base, BPW)], iidx, isem)
    h_w = pltpu.async_copy(wb_hbm, wb_v, usem)
    h_u.wait()
    h_i.wait()
    h_w.wait()

    ubufs = (u0, u1, u2)
    ibufs = (i0, i1, i2)

    def start(c):
        ub = ubufs[c % 3]
        ib = ibufs[c % 3]
        hu = pltpu.async_copy(ut_hbm.at[uidx.at[pl.ds(c * CHUNK, CHUNK)]], ub, usem)
        hi = pltpu.async_copy(it_hbm.at[iidx.at[pl.ds(c * CHUNK, CHUNK)]], ib, isem)
        return hu, hi

    pending = [start(0), start(1), start(2)]

    riota = lax.iota(jnp.int32, 16)
    ridx = [riota + (g * 16) for g in range(NGRP)]
    bvec = wb_v[D]

    for c in range(NCHUNK):
        hu, hi = pending[c]
        hu.wait()
        hi.wait()

        ub = ubufs[c % 3]
        ib = ibufs[c % 3]

        def dbody(d, accs, ub=ub, ib=ib):
            w_d = wb_v[d]
            col = (riota + d) & (D - 1)
            out = []
            for g in range(NGRP):
                ug = plsc.load_gather(ub, [ridx[g], col])
                ig = plsc.load_gather(ib, [ridx[g], col])
                pb = plsc.bitcast(ug * ig, jnp.int32)
                pb = (pb + 0x8000) & jnp.int32(-65536)
                p = plsc.bitcast(pb, jnp.float32)
                out.append(accs[g] + p * w_d)
            return tuple(out)

        accs = lax.fori_loop(
            0, D, dbody,
            tuple(jnp.zeros((16,), jnp.float32) for _ in range(NGRP)))

        for g in range(NGRP):
            out_v[pl.ds(c * CHUNK + g * 16, 16)] = accs[g] + bvec

        if c + 3 < NCHUNK:
            pending.append(start(c + 3))

    pltpu.sync_copy(out_v, out_hbm.at[pl.ds(base, BPW)])


@jax.jit
def kernel(users, items, user_table, item_table, W, b):
    users_r = users.astype(jnp.int32)
    items_r = items.astype(jnp.int32)
    wi = lax.bitcast_convert_type(W.astype(jnp.float32), jnp.int32)
    wi = (wi + 0x7FFF + (lax.shift_right_logical(wi, 16) & 1)) & jnp.int32(-65536)
    w_rounded = lax.bitcast_convert_type(wi, jnp.float32)[:, 0]
    w2 = jnp.concatenate([w_rounded, w_rounded])
    wsk = jnp.stack([w2[l:l + D] for l in range(16)], axis=1)
    wb = jnp.concatenate([wsk, jnp.broadcast_to(b, (1, 16))]).astype(jnp.float32)

    mesh = plsc.VectorSubcoreMesh(core_axis_name="c", subcore_axis_name="s")
    out = pl.kernel(
        _mf_body,
        mesh=mesh,
        compiler_params=pltpu.CompilerParams(needs_layout_passes=False),
        out_type=jax.ShapeDtypeStruct((B,), jnp.float32),
        scratch_types=[
            pltpu.VMEM((BPW,), jnp.int32),
            pltpu.VMEM((BPW,), jnp.int32),
            pltpu.VMEM((CHUNK, D), jnp.float32),
            pltpu.VMEM((CHUNK, D), jnp.float32),
            pltpu.VMEM((CHUNK, D), jnp.float32),
            pltpu.VMEM((CHUNK, D), jnp.float32),
            pltpu.VMEM((CHUNK, D), jnp.float32),
            pltpu.VMEM((CHUNK, D), jnp.float32),
            pltpu.VMEM((D + 1, 16), jnp.float32),
            pltpu.VMEM((BPW,), jnp.float32),
            pltpu.SemaphoreType.DMA,
            pltpu.SemaphoreType.DMA,
        ],
    )(users_r, items_r, user_table, item_table, wb)
    return out.reshape(B, 1)


# speedup vs baseline: 1.5757x; 1.0180x over previous
import jax
import jax.numpy as jnp
from jax import lax
from jax.experimental import pallas as pl
from jax.experimental.pallas import tpu as pltpu
from jax.experimental.pallas import tpu_sc as plsc

B = 16384
D = 128
NC = 2
NS = 16
NW = NC * NS
BPW = B // NW
CHUNK = 64
NCHUNK = BPW // CHUNK
NGRP = CHUNK // 16
NBUF = 4


def _mf_body(users_hbm, items_hbm, ut_hbm, it_hbm, wb_hbm, out_hbm,
             uidx, iidx, u0, u1, u2, u3, i0, i1, i2, i3, wb_v, out_v,
             usem, isem, osem):
    wid = lax.axis_index("s") * NC + lax.axis_index("c")
    base = wid * BPW

    h_u = pltpu.async_copy(users_hbm.at[pl.ds(base, BPW)], uidx, usem)
    h_i = pltpu.async_copy(items_hbm.at[pl.ds(base, BPW)], iidx, isem)
    h_w = pltpu.async_copy(wb_hbm, wb_v, usem)
    h_u.wait()
    h_i.wait()
    h_w.wait()

    ubufs = (u0, u1, u2, u3)
    ibufs = (i0, i1, i2, i3)

    def start(c):
        ub = ubufs[c % NBUF]
        ib = ibufs[c % NBUF]
        hu = pltpu.async_copy(ut_hbm.at[uidx.at[pl.ds(c * CHUNK, CHUNK)]], ub, usem)
        hi = pltpu.async_copy(it_hbm.at[iidx.at[pl.ds(c * CHUNK, CHUNK)]], ib, isem)
        return hu, hi

    pending = [start(0), start(1), start(2), start(3)]
    out_pending = []

    riota = lax.iota(jnp.int32, 16)
    ridx = [riota + (g * 16) for g in range(NGRP)]
    bvec = wb_v[D]

    for c in range(NCHUNK):
        hu, hi = pending[c]
        hu.wait()
        hi.wait()

        ub = ubufs[c % NBUF]
        ib = ibufs[c % NBUF]

        def dbody(d, accs, ub=ub, ib=ib):
            w_d = wb_v[d]
            col = (riota + d) & (D - 1)
            out = []
            for g in range(NGRP):
                ug = plsc.load_gather(ub, [ridx[g], col])
                ig = plsc.load_gather(ib, [ridx[g], col])
                pb = plsc.bitcast(ug * ig, jnp.int32)
                pb = (pb + 0x8000) & jnp.int32(-65536)
                p = plsc.bitcast(pb, jnp.float32)
                out.append(accs[g] + p * w_d)
            return tuple(out)

        accs = lax.fori_loop(
            0, D, dbody,
            tuple(jnp.zeros((16,), jnp.float32) for _ in range(NGRP)))

        for g in range(NGRP):
            out_v[pl.ds(c * CHUNK + g * 16, 16)] = accs[g] + bvec
        out_pending.append(pltpu.async_copy(
            out_v.at[pl.ds(c * CHUNK, CHUNK)],
            out_hbm.at[pl.ds(base + c * CHUNK, CHUNK)], osem))

        if c + NBUF < NCHUNK:
            pending.append(start(c + NBUF))

    for h in out_pending:
        h.wait()


@jax.jit
def kernel(users, items, user_table, item_table, W, b):
    users_r = users.astype(jnp.int32)
    items_r = items.astype(jnp.int32)
    wi = lax.bitcast_convert_type(W.astype(jnp.float32), jnp.int32)
    wi = (wi + 0x7FFF + (lax.shift_right_logical(wi, 16) & 1)) & jnp.int32(-65536)
    w_rounded = lax.bitcast_convert_type(wi, jnp.float32)[:, 0]
    w2 = jnp.concatenate([w_rounded, w_rounded])
    wsk = jnp.stack([w2[l:l + D] for l in range(16)], axis=1)
    wb = jnp.concatenate([wsk, jnp.broadcast_to(b, (1, 16))]).astype(jnp.float32)

    mesh = plsc.VectorSubcoreMesh(core_axis_name="c", subcore_axis_name="s")
    out = pl.kernel(
        _mf_body,
        mesh=mesh,
        compiler_params=pltpu.CompilerParams(needs_layout_passes=False),
        out_type=jax.ShapeDtypeStruct((B,), jnp.float32),
        scratch_types=[
            pltpu.VMEM((BPW,), jnp.int32),
            pltpu.VMEM((BPW,), jnp.int32),
            pltpu.VMEM((CHUNK, D), jnp.float32),
            pltpu.VMEM((CHUNK, D), jnp.float32),
            pltpu.VMEM((CHUNK, D), jnp.float32),
            pltpu.VMEM((CHUNK, D), jnp.float32),
            pltpu.VMEM((CHUNK, D), jnp.float32),
            pltpu.VMEM((CHUNK, D), jnp.float32),
            pltpu.VMEM((CHUNK, D), jnp.float32),
            pltpu.VMEM((CHUNK, D), jnp.float32),
            pltpu.VMEM((D + 1, 16), jnp.float32),
            pltpu.VMEM((BPW,), jnp.float32),
            pltpu.SemaphoreType.DMA,
            pltpu.SemaphoreType.DMA,
            pltpu.SemaphoreType.DMA,
        ],
    )(users_r, items_r, user_table, item_table, wb)
    return out.reshape(B, 1)


# speedup vs baseline: 1.6536x; 1.0494x over previous
import jax
import jax.numpy as jnp
from jax import lax
from jax.experimental import pallas as pl
from jax.experimental.pallas import tpu as pltpu
from jax.experimental.pallas import tpu_sc as plsc

B = 16384
D = 128
NC = 2
NS = 16
NW = NC * NS
BPW = B // NW
CHUNK = 64
NCHUNK = BPW // CHUNK
NGRP = CHUNK // 16
NBUF = 4


def _mf_body(users_hbm, items_hbm, ut_hbm, it_hbm, wb_hbm, out_hbm,
             uidx, iidx, u0, u1, u2, u3, i0, i1, i2, i3, wb_v, out_v,
             usem, isem, osem):
    wid = lax.axis_index("s") * NC + lax.axis_index("c")
    base = wid * BPW

    h_u = pltpu.async_copy(users_hbm.at[pl.ds(base, BPW)], uidx, usem)
    h_i = pltpu.async_copy(items_hbm.at[pl.ds(base, BPW)], iidx, isem)
    h_w = pltpu.async_copy(wb_hbm, wb_v, usem)
    h_u.wait()
    h_i.wait()
    h_w.wait()

    ubufs = (u0, u1, u2, u3)
    ibufs = (i0, i1, i2, i3)

    def start(c):
        ub = ubufs[c % NBUF]
        ib = ibufs[c % NBUF]
        hu = pltpu.async_copy(ut_hbm.at[uidx.at[pl.ds(c * CHUNK, CHUNK)]], ub, usem)
        hi = pltpu.async_copy(it_hbm.at[iidx.at[pl.ds(c * CHUNK, CHUNK)]], ib, isem)
        return hu, hi

    pending = [start(0), start(1), start(2), start(3)]
    out_pending = []

    riota = lax.iota(jnp.int32, 16)
    ridx = [riota + (g * 16) for g in range(NGRP)]
    bvec = wb_v[pl.ds(D, 16)]

    for c in range(NCHUNK):
        hu, hi = pending[c]
        hu.wait()
        hi.wait()

        ub = ubufs[c % NBUF]
        ib = ibufs[c % NBUF]

        def dbody(d, accs, ub=ub, ib=ib):
            col = (riota + d) & (D - 1)
            w_d = plsc.load_gather(wb_v, [col])
            out = []
            for g in range(NGRP):
                ug = plsc.load_gather(ub, [ridx[g], col])
                ig = plsc.load_gather(ib, [ridx[g], col])
                pb = plsc.bitcast(ug * ig, jnp.int32)
                pb = (pb + 0x8000) & jnp.int32(-65536)
                p = plsc.bitcast(pb, jnp.float32)
                out.append(accs[g] + p * w_d)
            return tuple(out)

        accs = lax.fori_loop(
            0, D, dbody,
            tuple(jnp.zeros((16,), jnp.float32) for _ in range(NGRP)))

        for g in range(NGRP):
            out_v[pl.ds(c * CHUNK + g * 16, 16)] = accs[g] + bvec
        out_pending.append(pltpu.async_copy(
            out_v.at[pl.ds(c * CHUNK, CHUNK)],
            out_hbm.at[pl.ds(base + c * CHUNK, CHUNK)], osem))

        if c + NBUF < NCHUNK:
            pending.append(start(c + NBUF))

    for h in out_pending:
        h.wait()


@jax.jit
def kernel(users, items, user_table, item_table, W, b):
    users_r = users.astype(jnp.int32)
    items_r = items.astype(jnp.int32)
    wi = lax.bitcast_convert_type(W.astype(jnp.float32), jnp.int32)
    wi = (wi + 0x7FFF + (lax.shift_right_logical(wi, 16) & 1)) & jnp.int32(-65536)
    w_rounded = lax.bitcast_convert_type(wi, jnp.float32)[:, 0]
    wb = jnp.concatenate([w_rounded, jnp.broadcast_to(b, (16,))]).astype(jnp.float32)

    mesh = plsc.VectorSubcoreMesh(core_axis_name="c", subcore_axis_name="s")
    out = pl.kernel(
        _mf_body,
        mesh=mesh,
        compiler_params=pltpu.CompilerParams(needs_layout_passes=False),
        out_type=jax.ShapeDtypeStruct((B,), jnp.float32),
        scratch_types=[
            pltpu.VMEM((BPW,), jnp.int32),
            pltpu.VMEM((BPW,), jnp.int32),
            pltpu.VMEM((CHUNK, D), jnp.float32),
            pltpu.VMEM((CHUNK, D), jnp.float32),
            pltpu.VMEM((CHUNK, D), jnp.float32),
            pltpu.VMEM((CHUNK, D), jnp.float32),
            pltpu.VMEM((CHUNK, D), jnp.float32),
            pltpu.VMEM((CHUNK, D), jnp.float32),
            pltpu.VMEM((CHUNK, D), jnp.float32),
            pltpu.VMEM((CHUNK, D), jnp.float32),
            pltpu.VMEM((D + 16,), jnp.float32),
            pltpu.VMEM((BPW,), jnp.float32),
            pltpu.SemaphoreType.DMA,
            pltpu.SemaphoreType.DMA,
            pltpu.SemaphoreType.DMA,
        ],
    )(users_r, items_r, user_table, item_table, wb)
    return out.reshape(B, 1)
